# baseline (speedup 1.0000x reference)
import jax
import jax.numpy as jnp
from jax import lax
from jax.experimental import pallas as pl
from jax.experimental.pallas import tpu as pltpu
from jax.experimental.pallas import tpu_sc as plsc

MAX_LEN = 8192
D_MODEL = 768
LANES = 16
NUM_CORES = 2
NUM_SUBCORES = 16
NUM_WORKERS = NUM_CORES * NUM_SUBCORES
ROWS_PER_WORKER = MAX_LEN // NUM_WORKERS
CHUNK = 16
NUM_CHUNKS = ROWS_PER_WORKER // CHUNK
NBUF = 4
VECS_PER_ROW = D_MODEL // LANES


def _body(hours_hbm, days_hbm, pe_hbm, htab_hbm, dtab_hbm, out_hbm,
          hidx_v, didx_v, htab_v, dtab_v, bufs, sem_in, sem_out,
          stg0, stg1, stg2, stg3):
    wid = lax.axis_index("s") * NUM_CORES + lax.axis_index("c")
    base = wid * ROWS_PER_WORKER

    stage = [
        pltpu.async_copy(hours_hbm.at[pl.ds(base, ROWS_PER_WORKER)],
                         hidx_v, stg0),
        pltpu.async_copy(days_hbm.at[pl.ds(base, ROWS_PER_WORKER)],
                         didx_v, stg1),
        pltpu.async_copy(htab_hbm, htab_v, stg2),
        pltpu.async_copy(dtab_hbm, dtab_v, stg3),
    ]

    def in_copy(c, slot, start):
        mk = pltpu.async_copy if start else pltpu.make_async_copy
        return mk(pe_hbm.at[pl.ds(base + c * CHUNK, CHUNK)], bufs.at[slot],
                  sem_in.at[slot])

    def out_copy(c, slot, start):
        mk = pltpu.async_copy if start else pltpu.make_async_copy
        return mk(bufs.at[slot], out_hbm.at[pl.ds(base + c * CHUNK, CHUNK)],
                  sem_out.at[slot])

    for k in range(NBUF - 1):
        in_copy(k, k, start=True)
    for cp in stage:
        cp.wait()

    def chunk_body(t, carry):
        slot = lax.rem(t, NBUF)
        in_copy(t, slot, start=False).wait()
        off = pl.multiple_of(t * CHUNK, CHUNK)
        hvec = hidx_v[pl.ds(off, LANES)]
        dvec = didx_v[pl.ds(off, LANES)]
        for r in range(CHUNK):
            h = hvec[r]
            d = dvec[r]

            @plsc.parallel_loop(0, VECS_PER_ROW, 1, unroll=3)
            def vec_body(j, _r=r, _h=h, _d=d):
                s = pl.ds(pl.multiple_of(j * LANES, LANES), LANES)
                bufs[slot, _r, s] = (bufs[slot, _r, s] + htab_v[_h, s]
                                     + dtab_v[_d, s])
        out_copy(t, slot, start=True)
        nslot = lax.rem(slot + NBUF - 1, NBUF)

        @pl.when(t >= 1)
        def _wait_prev():
            out_copy(t - 1, nslot, start=False).wait()

        @pl.when(t + NBUF - 1 < NUM_CHUNKS)
        def _start_next():
            in_copy(t + NBUF - 1, nslot, start=True)
        return carry

    lax.fori_loop(0, NUM_CHUNKS, chunk_body, 0)
    out_copy(NUM_CHUNKS - 1, (NUM_CHUNKS - 1) % NBUF, start=False).wait()


@jax.jit
def _temporal_embedding(hours, days, pe, hour_embedding, day_embedding):
    mesh = plsc.VectorSubcoreMesh(
        core_axis_name="c", subcore_axis_name="s",
        num_cores=NUM_CORES, num_subcores=NUM_SUBCORES)
    return pl.kernel(
        _body,
        out_type=jax.ShapeDtypeStruct((MAX_LEN, D_MODEL), jnp.float32),
        mesh=mesh,
        scratch_types=[
            pltpu.VMEM((ROWS_PER_WORKER,), jnp.int32),
            pltpu.VMEM((ROWS_PER_WORKER,), jnp.int32),
            pltpu.VMEM((24, D_MODEL), jnp.float32),
            pltpu.VMEM((7, D_MODEL), jnp.float32),
            pltpu.VMEM((NBUF, CHUNK, D_MODEL), jnp.float32),
            pltpu.SemaphoreType.DMA((NBUF,)),
            pltpu.SemaphoreType.DMA((NBUF,)),
            pltpu.SemaphoreType.DMA,
            pltpu.SemaphoreType.DMA,
            pltpu.SemaphoreType.DMA,
            pltpu.SemaphoreType.DMA,
        ],
    )(hours, days, pe, hour_embedding, day_embedding)


def kernel(hours, days, pe, hour_embedding, day_embedding):
    return _temporal_embedding(hours, days, pe, hour_embedding, day_embedding)


# speedup vs baseline: 1.1696x; 1.1696x over previous
import jax
import jax.numpy as jnp
from jax import lax
from jax.experimental import pallas as pl
from jax.experimental.pallas import tpu as pltpu
from jax.experimental.pallas import tpu_sc as plsc

MAX_LEN = 8192
D_MODEL = 768
LANES = 16
NUM_CORES = 2
NUM_SUBCORES = 16
NUM_WORKERS = NUM_CORES * NUM_SUBCORES
ROWS_PER_WORKER = MAX_LEN // NUM_WORKERS
CHUNK = 16
NUM_CHUNKS = ROWS_PER_WORKER // CHUNK
NBUF = 4
VECS_PER_ROW = D_MODEL // LANES


def _body(hours_hbm, days_hbm, pe_hbm, htab_hbm, dtab_hbm, out_hbm,
          hidx_v, didx_v, htab_v, dtab_v, bufs, sem_in, sem_out,
          stg0, stg1, stg2, stg3):
    wid = lax.axis_index("s") * NUM_CORES + lax.axis_index("c")
    base = wid * ROWS_PER_WORKER

    stage = [
        pltpu.async_copy(hours_hbm.at[pl.ds(base, ROWS_PER_WORKER)],
                         hidx_v, stg0),
        pltpu.async_copy(days_hbm.at[pl.ds(base, ROWS_PER_WORKER)],
                         didx_v, stg1),
        pltpu.async_copy(htab_hbm, htab_v, stg2),
        pltpu.async_copy(dtab_hbm, dtab_v, stg3),
    ]

    def in_copy(c, slot, start):
        mk = pltpu.async_copy if start else pltpu.make_async_copy
        return mk(pe_hbm.at[pl.ds(base + c * CHUNK, CHUNK)], bufs.at[slot],
                  sem_in.at[slot])

    def out_copy(c, slot, start):
        mk = pltpu.async_copy if start else pltpu.make_async_copy
        return mk(bufs.at[slot], out_hbm.at[pl.ds(base + c * CHUNK, CHUNK)],
                  sem_out.at[slot])

    for k in range(NBUF - 1):
        in_copy(k, k, start=True)
    for cp in stage:
        cp.wait()

    def chunk_body(t, carry):
        slot = lax.rem(t, NBUF)
        in_copy(t, slot, start=False).wait()
        off = pl.multiple_of(t * CHUNK, CHUNK)
        hvec = hidx_v[pl.ds(off, LANES)]
        dvec = didx_v[pl.ds(off, LANES)]
        for r in range(0, CHUNK, 2):
            h0, d0 = hvec[r], dvec[r]
            h1, d1 = hvec[r + 1], dvec[r + 1]

            @plsc.parallel_loop(0, VECS_PER_ROW, 1, unroll=2)
            def vec_body(j, _r=r, _h0=h0, _d0=d0, _h1=h1, _d1=d1):
                s = pl.ds(pl.multiple_of(j * LANES, LANES), LANES)
                bufs[slot, _r, s] = (bufs[slot, _r, s] + htab_v[_h0, s]
                                     + dtab_v[_d0, s])
                bufs[slot, _r + 1, s] = (bufs[slot, _r + 1, s]
                                         + htab_v[_h1, s]
                                         + dtab_v[_d1, s])
        out_copy(t, slot, start=True)
        nslot = lax.rem(slot + NBUF - 1, NBUF)

        @pl.when(t >= 1)
        def _wait_prev():
            out_copy(t - 1, nslot, start=False).wait()

        @pl.when(t + NBUF - 1 < NUM_CHUNKS)
        def _start_next():
            in_copy(t + NBUF - 1, nslot, start=True)
        return carry

    lax.fori_loop(0, NUM_CHUNKS, chunk_body, 0)
    out_copy(NUM_CHUNKS - 1, (NUM_CHUNKS - 1) % NBUF, start=False).wait()


@jax.jit
def _temporal_embedding(hours, days, pe, hour_embedding, day_embedding):
    mesh = plsc.VectorSubcoreMesh(
        core_axis_name="c", subcore_axis_name="s",
        num_cores=NUM_CORES, num_subcores=NUM_SUBCORES)
    return pl.kernel(
        _body,
        out_type=jax.ShapeDtypeStruct((MAX_LEN, D_MODEL), jnp.float32),
        mesh=mesh,
        scratch_types=[
            pltpu.VMEM((ROWS_PER_WORKER,), jnp.int32),
            pltpu.VMEM((ROWS_PER_WORKER,), jnp.int32),
            pltpu.VMEM((24, D_MODEL), jnp.float32),
            pltpu.VMEM((7, D_MODEL), jnp.float32),
            pltpu.VMEM((NBUF, CHUNK, D_MODEL), jnp.float32),
            pltpu.SemaphoreType.DMA((NBUF,)),
            pltpu.SemaphoreType.DMA((NBUF,)),
            pltpu.SemaphoreType.DMA,
            pltpu.SemaphoreType.DMA,
            pltpu.SemaphoreType.DMA,
            pltpu.SemaphoreType.DMA,
        ],
    )(hours, days, pe, hour_embedding, day_embedding)


def kernel(hours, days, pe, hour_embedding, day_embedding):
    return _temporal_embedding(hours, days, pe, hour_embedding, day_embedding)


# speedup vs baseline: 1.1826x; 1.0111x over previous
import jax
import jax.numpy as jnp
from jax import lax
from jax.experimental import pallas as pl
from jax.experimental.pallas import tpu as pltpu
from jax.experimental.pallas import tpu_sc as plsc

MAX_LEN = 8192
D_MODEL = 768
LANES = 16
NUM_CORES = 2
NUM_SUBCORES = 16
NUM_WORKERS = NUM_CORES * NUM_SUBCORES
ROWS_PER_WORKER = MAX_LEN // NUM_WORKERS
CHUNK = 16
NUM_CHUNKS = ROWS_PER_WORKER // CHUNK
NBUF = 4
VECS_PER_ROW = D_MODEL // LANES


def _body(hours_hbm, days_hbm, pe_hbm, htab_hbm, dtab_hbm, out_hbm,
          hidx_v, didx_v, htab_v, dtab_v, bufs, sem_in, sem_out,
          stg0, stg1, stg2, stg3):
    wid = lax.axis_index("s") * NUM_CORES + lax.axis_index("c")
    base = wid * ROWS_PER_WORKER

    stage = [
        pltpu.async_copy(hours_hbm.at[pl.ds(base, ROWS_PER_WORKER)],
                         hidx_v, stg0),
        pltpu.async_copy(days_hbm.at[pl.ds(base, ROWS_PER_WORKER)],
                         didx_v, stg1),
        pltpu.async_copy(htab_hbm, htab_v, stg2),
        pltpu.async_copy(dtab_hbm, dtab_v, stg3),
    ]

    def in_copy(c, slot, start):
        mk = pltpu.async_copy if start else pltpu.make_async_copy
        return mk(pe_hbm.at[pl.ds(base + c * CHUNK, CHUNK)], bufs.at[slot],
                  sem_in.at[slot])

    def out_copy(c, slot, start):
        mk = pltpu.async_copy if start else pltpu.make_async_copy
        return mk(bufs.at[slot], out_hbm.at[pl.ds(base + c * CHUNK, CHUNK)],
                  sem_out.at[slot])

    for k in range(NBUF - 1):
        in_copy(k, k, start=True)
    for cp in stage:
        cp.wait()

    def chunk_body(t, carry):
        slot = lax.rem(t, NBUF)
        in_copy(t, slot, start=False).wait()
        off = pl.multiple_of(t * CHUNK, CHUNK)
        hvec = hidx_v[pl.ds(off, LANES)]
        dvec = didx_v[pl.ds(off, LANES)]
        for r in range(0, CHUNK, 4):
            hh = [hvec[r + i] for i in range(4)]
            dd = [dvec[r + i] for i in range(4)]

            @plsc.parallel_loop(0, VECS_PER_ROW, 1, unroll=1)
            def vec_body(j, _r=r, _hh=hh, _dd=dd):
                s = pl.ds(pl.multiple_of(j * LANES, LANES), LANES)
                for i in range(4):
                    bufs[slot, _r + i, s] = (bufs[slot, _r + i, s]
                                             + htab_v[_hh[i], s]
                                             + dtab_v[_dd[i], s])
        out_copy(t, slot, start=True)
        nslot = lax.rem(slot + NBUF - 1, NBUF)

        @pl.when(t >= 1)
        def _wait_prev():
            out_copy(t - 1, nslot, start=False).wait()

        @pl.when(t + NBUF - 1 < NUM_CHUNKS)
        def _start_next():
            in_copy(t + NBUF - 1, nslot, start=True)
        return carry

    lax.fori_loop(0, NUM_CHUNKS, chunk_body, 0)
    out_copy(NUM_CHUNKS - 1, (NUM_CHUNKS - 1) % NBUF, start=False).wait()


@jax.jit
def _temporal_embedding(hours, days, pe, hour_embedding, day_embedding):
    mesh = plsc.VectorSubcoreMesh(
        core_axis_name="c", subcore_axis_name="s",
        num_cores=NUM_CORES, num_subcores=NUM_SUBCORES)
    return pl.kernel(
        _body,
        out_type=jax.ShapeDtypeStruct((MAX_LEN, D_MODEL), jnp.float32),
        mesh=mesh,
        scratch_types=[
            pltpu.VMEM((ROWS_PER_WORKER,), jnp.int32),
            pltpu.VMEM((ROWS_PER_WORKER,), jnp.int32),
            pltpu.VMEM((24, D_MODEL), jnp.float32),
            pltpu.VMEM((7, D_MODEL), jnp.float32),
            pltpu.VMEM((NBUF, CHUNK, D_MODEL), jnp.float32),
            pltpu.SemaphoreType.DMA((NBUF,)),
            pltpu.SemaphoreType.DMA((NBUF,)),
            pltpu.SemaphoreType.DMA,
            pltpu.SemaphoreType.DMA,
            pltpu.SemaphoreType.DMA,
            pltpu.SemaphoreType.DMA,
        ],
    )(hours, days, pe, hour_embedding, day_embedding)


def kernel(hours, days, pe, hour_embedding, day_embedding):
    return _temporal_embedding(hours, days, pe, hour_embedding, day_embedding)
